# baseline (speedup 1.0000x reference)
import jax
import jax.numpy as jnp
from jax.experimental import pallas as pl
from jax.experimental.pallas import tpu as pltpu

_NH = 1024
_DT = 0.1
_DELAY = 512
_K = 0.1

_ROWS = 2048
_COLS = 32768


def _body(ts_ref, bufw_ref, rowa_ref, rowb_ref, rowx_ref, w_ref,
          outb_ref, outnx_ref):
    ts = ts_ref[0]
    ra = (_NH + ts - _DELAY) % 8
    rb = (_NH + ts + 1 - _DELAY) % 8
    rx = (_NH + ts) % 8
    rp = (_NH + ts + 1) % 8
    x = rowx_ref[rx, :]
    a = rowa_ref[ra, :]
    b = rowb_ref[rb, :]
    w = w_ref[...]
    d1 = -x + _K * jnp.tanh(a)
    xi = x + _DT * d1 + w
    d2 = -xi + _K * jnp.tanh(b)
    nx = x + _DT * 0.5 * (d1 + d2) + w
    outnx_ref[...] = nx
    outb_ref[...] = bufw_ref[...]
    outb_ref[rp, :] = nx


def kernel(buf, dWt, t):
    ts = t[0, 0:1].astype(jnp.int32)
    grid_spec = pltpu.PrefetchScalarGridSpec(
        num_scalar_prefetch=1,
        grid=(1,),
        in_specs=[
            pl.BlockSpec((8, _COLS), lambda i, ts: ((_NH + ts[0] + 1) // 8, 0)),
            pl.BlockSpec((8, _COLS), lambda i, ts: ((_NH + ts[0] - _DELAY) // 8, 0)),
            pl.BlockSpec((8, _COLS), lambda i, ts: ((_NH + ts[0] + 1 - _DELAY) // 8, 0)),
            pl.BlockSpec((8, _COLS), lambda i, ts: ((_NH + ts[0]) // 8, 0)),
            pl.BlockSpec((_COLS,), lambda i, ts: (0,)),
        ],
        out_specs=[
            pl.BlockSpec((8, _COLS), lambda i, ts: ((_NH + ts[0] + 1) // 8, 0)),
            pl.BlockSpec((_COLS,), lambda i, ts: (0,)),
        ],
    )
    buf2, nx = pl.pallas_call(
        _body,
        grid_spec=grid_spec,
        out_shape=[
            jax.ShapeDtypeStruct((_ROWS, _COLS), jnp.float32),
            jax.ShapeDtypeStruct((_COLS,), jnp.float32),
        ],
        input_output_aliases={1: 0},
    )(ts, buf, buf, buf, buf, dWt)
    return (buf2, nx)


# speedup vs baseline: 1.0116x; 1.0116x over previous
import jax
import jax.numpy as jnp
from jax.experimental import pallas as pl
from jax.experimental.pallas import tpu as pltpu

_NH = 1024
_DT = 0.1
_DELAY = 512
_K = 0.1

_ROWS = 2048
_COLS = 32768
_R = 64
_GRID = _ROWS // _R


def _body(ts_ref, buf_ref, rowa_ref, rowb_ref, rowx_ref, w_ref,
          outb_ref, outnx_ref, nx_ref):
    ts = ts_ref[0]
    i = pl.program_id(0)
    outb_ref[...] = buf_ref[...]

    @pl.when(i == 0)
    def _compute():
        x = rowx_ref[(_NH + ts) % 8, :]
        a = rowa_ref[(_NH + ts - _DELAY) % 8, :]
        b = rowb_ref[(_NH + ts + 1 - _DELAY) % 8, :]
        w = w_ref[...]
        d1 = -x + _K * jnp.tanh(a)
        xi = x + _DT * d1 + w
        d2 = -xi + _K * jnp.tanh(b)
        nx = x + _DT * 0.5 * (d1 + d2) + w
        outnx_ref[...] = nx
        nx_ref[...] = nx

    @pl.when(i == (_NH + ts + 1) // _R)
    def _patch():
        outb_ref[(_NH + ts + 1) % _R, :] = nx_ref[...]


def kernel(buf, dWt, t):
    ts = t[0, 0:1].astype(jnp.int32)
    grid_spec = pltpu.PrefetchScalarGridSpec(
        num_scalar_prefetch=1,
        grid=(_GRID,),
        in_specs=[
            pl.BlockSpec((_R, _COLS), lambda i, ts: (i, 0)),
            pl.BlockSpec((8, _COLS), lambda i, ts: ((_NH + ts[0] - _DELAY) // 8, 0)),
            pl.BlockSpec((8, _COLS), lambda i, ts: ((_NH + ts[0] + 1 - _DELAY) // 8, 0)),
            pl.BlockSpec((8, _COLS), lambda i, ts: ((_NH + ts[0]) // 8, 0)),
            pl.BlockSpec((_COLS,), lambda i, ts: (0,)),
        ],
        out_specs=[
            pl.BlockSpec((_R, _COLS), lambda i, ts: (i, 0)),
            pl.BlockSpec((_COLS,), lambda i, ts: (0,)),
        ],
        scratch_shapes=[pltpu.VMEM((_COLS,), jnp.float32)],
    )
    buf2, nx = pl.pallas_call(
        _body,
        grid_spec=grid_spec,
        out_shape=[
            jax.ShapeDtypeStruct((_ROWS, _COLS), jnp.float32),
            jax.ShapeDtypeStruct((_COLS,), jnp.float32),
        ],
    )(ts, buf, buf, buf, buf, dWt)
    return (buf2, nx)
